# baseline (speedup 1.0000x reference)
import functools

import numpy as np
import jax
import jax.numpy as jnp
from jax import lax
from jax.experimental import pallas as pl
from jax.experimental.pallas import tpu as pltpu
from jax.experimental.pallas import tpu_sc as plsc


def _sinusoidal_pe(seq_len, d_model):
    pos = np.arange(seq_len, dtype=np.float32)[:, None]
    div = np.exp(np.arange(0, d_model, 2, dtype=np.float32) * (-np.log(10000.0) / d_model))
    pe = np.zeros((seq_len, d_model), dtype=np.float32)
    pe[:, 0::2] = np.sin(pos * div)
    pe[:, 1::2] = np.cos(pos * div)
    return pe


def _expand_perm(e):
    w = np.arange(e // 2)
    perm = np.empty(e, dtype=np.int64)
    perm[0::2] = 32 * (w // 16) + (w % 16)
    perm[1::2] = 32 * (w // 16) + 16 + (w % 16)
    return perm


def _mm_body(x_ref, w_ref, o_ref):
    o_ref[...] = jnp.dot(
        x_ref[...], w_ref[...], preferred_element_type=jnp.float32
    ).astype(jnp.bfloat16)


def _table_transform(table, w_top):
    v, e = table.shape
    bm = 2000
    assert v % bm == 0
    return pl.pallas_call(
        _mm_body,
        grid=(v // bm,),
        in_specs=[
            pl.BlockSpec((bm, e), lambda i: (i, 0)),
            pl.BlockSpec((e, e), lambda i: (0, 0)),
        ],
        out_specs=pl.BlockSpec((bm, e), lambda i: (i, 0)),
        out_shape=jax.ShapeDtypeStruct((v, e), jnp.bfloat16),
    )(table, w_top)


def _pos_body(pe_ref, w_ref, b_ref, o_ref):
    o_ref[...] = (
        jnp.dot(pe_ref[...], w_ref[...], preferred_element_type=jnp.float32)
        + b_ref[...]
    )


def _pos_transform(pe, w_bot, b):
    s, e = pe.shape
    return pl.pallas_call(
        _pos_body,
        in_specs=[
            pl.BlockSpec((s, e), lambda: (0, 0)),
            pl.BlockSpec((e, e), lambda: (0, 0)),
            pl.BlockSpec((1, e), lambda: (0, 0)),
        ],
        out_specs=pl.BlockSpec((s, e), lambda: (0, 0)),
        out_shape=jax.ShapeDtypeStruct((s, e), jnp.float32),
    )(pe, w_bot, b.reshape(1, e))


def _sc_gather(idx, table2w, pos2, B, S, E):
    info = plsc.get_sparse_core_info()
    NC, NS = info.num_cores, info.num_subcores
    NW = NC * NS
    assert B % NW == 0
    bpw = B // NW
    EW = E // 2

    n0 = 128
    n1 = S - n0

    mesh = plsc.VectorSubcoreMesh(core_axis_name="c", subcore_axis_name="s")

    @functools.partial(
        pl.kernel,
        mesh=mesh,
        compiler_params=pltpu.CompilerParams(
            needs_layout_passes=False, use_tc_tiling_on_sc=False
        ),
        out_type=jax.ShapeDtypeStruct((B * S, E), jnp.float32),
        scratch_types=[
            pltpu.VMEM((bpw * S,), jnp.int32),
            pltpu.VMEM((S, E), jnp.float32),
            pltpu.VMEM((S, E), jnp.bfloat16),
            pltpu.VMEM((S, E), jnp.bfloat16),
            pltpu.VMEM((S, E), jnp.float32),
            pltpu.VMEM((S, E), jnp.float32),
            pltpu.SemaphoreType.DMA,
            pltpu.SemaphoreType.DMA,
        ],
    )
    def k(idx_hbm, table2_hbm, pos2_hbm, out_hbm, idx_v, pos_v, g0, g1, o0, o1, sem_g, sem_w):
        wid = lax.axis_index("s") * NC + lax.axis_index("c")
        base_b = wid * bpw
        pltpu.sync_copy(pos2_hbm, pos_v)
        pltpu.sync_copy(idx_hbm.at[pl.ds(base_b * S, bpw * S)], idx_v)
        gbufs = (g0, g1)
        obufs = (o0, o1)

        def gather_descs(i, buf):
            off = i * S
            return (
                pltpu.make_async_copy(
                    table2_hbm.at[idx_v.at[pl.ds(off, n0)]], buf.at[pl.ds(0, n0)], sem_g
                ),
                pltpu.make_async_copy(
                    table2_hbm.at[idx_v.at[pl.ds(off + n0, n1)]],
                    buf.at[pl.ds(n0, n1)],
                    sem_g,
                ),
            )

        def out_desc(i, buf):
            return pltpu.make_async_copy(
                buf, out_hbm.at[pl.ds((base_b + i) * S, S)], sem_w
            )

        def expand_add(gbuf, obuf):
            def rbody(r2, c2):
                for dr in range(2):
                    r = r2 * 2 + dr
                    for c in range(EW // 16):
                        bf = gbuf[r, pl.ds(c * 32, 32)]
                        lo, hi = plsc.unpack(bf, format=plsc.PackFormat.INTERLEAVED)
                        obuf[r, pl.ds(c * 32, 16)] = lo + pos_v[r, pl.ds(c * 32, 16)]
                        obuf[r, pl.ds(c * 32 + 16, 16)] = (
                            hi + pos_v[r, pl.ds(c * 32 + 16, 16)]
                        )
                return c2

            lax.fori_loop(0, S // 2, rbody, 0)

        for d in gather_descs(0, g0):
            d.start()

        def body(j, carry):
            for h in range(2):
                i = 2 * j + h
                gbuf = gbufs[h]
                ngbuf = gbufs[1 - h]
                obuf = obufs[h]
                for d in gather_descs(i, gbuf):
                    d.wait()

                @pl.when(i + 1 < bpw)
                def _():
                    for d in gather_descs(i + 1, ngbuf):
                        d.start()

                @pl.when(i >= 2)
                def _():
                    out_desc(i - 2, obuf).wait()

                expand_add(gbuf, obuf)
                out_desc(i, obuf).start()
            return carry

        lax.fori_loop(0, bpw // 2, body, 0)
        out_desc(bpw - 2, o0).wait()
        out_desc(bpw - 1, o1).wait()

    return k(idx.reshape(B * S), table2w, pos2)


def kernel(pre_embedding, preembed_mask, embed_table, W, b):
    B, S = pre_embedding.shape
    V, E = embed_table.shape
    perm = _expand_perm(E)
    w_top = W[:E, :][:, perm]
    w_bot = W[E:, :]
    pe = jnp.asarray(_sinusoidal_pe(S, E))

    table2 = _table_transform(embed_table, w_top)
    pos2 = _pos_transform(pe, w_bot, b)
    idx = pre_embedding.astype(jnp.int32)
    out = _sc_gather(idx, table2, pos2, B, S, E)
    return (out.reshape(B, S, E), preembed_mask)


# speedup vs baseline: 1.9147x; 1.9147x over previous
import functools

import numpy as np
import jax
import jax.numpy as jnp
from jax import lax
from jax.experimental import pallas as pl
from jax.experimental.pallas import tpu as pltpu
from jax.experimental.pallas import tpu_sc as plsc


def _sinusoidal_pe(seq_len, d_model):
    pos = np.arange(seq_len, dtype=np.float32)[:, None]
    div = np.exp(np.arange(0, d_model, 2, dtype=np.float32) * (-np.log(10000.0) / d_model))
    pe = np.zeros((seq_len, d_model), dtype=np.float32)
    pe[:, 0::2] = np.sin(pos * div)
    pe[:, 1::2] = np.cos(pos * div)
    return pe


def _mm_body(x_ref, w_ref, o_ref):
    o_ref[...] = jnp.dot(x_ref[...], w_ref[...], preferred_element_type=jnp.float32)


def _table_transform(table, w_top):
    v, e = table.shape
    bm = 2000
    assert v % bm == 0
    return pl.pallas_call(
        _mm_body,
        grid=(v // bm,),
        in_specs=[
            pl.BlockSpec((bm, e), lambda i: (i, 0)),
            pl.BlockSpec((e, e), lambda i: (0, 0)),
        ],
        out_specs=pl.BlockSpec((bm, e), lambda i: (i, 0)),
        out_shape=jax.ShapeDtypeStruct((v, e), jnp.float32),
    )(table, w_top)


def _pos_body(pe_ref, w_ref, b_ref, o_ref):
    o_ref[...] = (
        jnp.dot(pe_ref[...], w_ref[...], preferred_element_type=jnp.float32)
        + b_ref[...]
    )


def _pos_transform(pe, w_bot, b):
    s, e = pe.shape
    return pl.pallas_call(
        _pos_body,
        in_specs=[
            pl.BlockSpec((s, e), lambda: (0, 0)),
            pl.BlockSpec((e, e), lambda: (0, 0)),
            pl.BlockSpec((1, e), lambda: (0, 0)),
        ],
        out_specs=pl.BlockSpec((s, e), lambda: (0, 0)),
        out_shape=jax.ShapeDtypeStruct((s, e), jnp.float32),
    )(pe, w_bot, b.reshape(1, e))


def _sc_gather(idx, table2, pos2, B, S, E):
    info = plsc.get_sparse_core_info()
    NC, NS = info.num_cores, info.num_subcores
    NW = NC * NS
    assert B % NW == 0
    bpw = B // NW
    NBUF = 3
    ntail = bpw % NBUF

    n0 = 128
    n1 = S - n0

    mesh = plsc.VectorSubcoreMesh(core_axis_name="c", subcore_axis_name="s")

    @functools.partial(
        pl.kernel,
        mesh=mesh,
        out_type=jax.ShapeDtypeStruct((B * S, E), jnp.float32),
        scratch_types=[
            pltpu.VMEM((bpw * S,), jnp.int32),
            pltpu.VMEM((S, E), jnp.float32),
            pltpu.VMEM((S, E), jnp.float32),
            pltpu.VMEM((S, E), jnp.float32),
            pltpu.VMEM((S, E), jnp.float32),
            pltpu.SemaphoreType.DMA,
            pltpu.SemaphoreType.DMA,
        ],
    )
    def k(idx_hbm, table2_hbm, pos2_hbm, out_hbm, idx_v, pos_v, b0, b1, b2, sem_g, sem_w):
        wid = lax.axis_index("s") * NC + lax.axis_index("c")
        base_b = wid * bpw
        pltpu.sync_copy(pos2_hbm, pos_v)
        pltpu.sync_copy(idx_hbm.at[pl.ds(base_b * S, bpw * S)], idx_v)
        bufs = (b0, b1, b2)

        def gather_descs(i, buf):
            off = i * S
            return (
                pltpu.make_async_copy(
                    table2_hbm.at[idx_v.at[pl.ds(off, n0)]], buf.at[pl.ds(0, n0)], sem_g
                ),
                pltpu.make_async_copy(
                    table2_hbm.at[idx_v.at[pl.ds(off + n0, n1)]],
                    buf.at[pl.ds(n0, n1)],
                    sem_g,
                ),
            )

        def out_desc(i, buf):
            return pltpu.make_async_copy(
                buf, out_hbm.at[pl.ds((base_b + i) * S, S)], sem_w
            )

        def add_pos(buf):
            def rbody(r4, c2):
                for dr in range(4):
                    r = r4 * 4 + dr
                    for c in range(E // 16):
                        plsc.addupdate(
                            buf.at[r, pl.ds(c * 16, 16)], pos_v[r, pl.ds(c * 16, 16)]
                        )
                return c2

            lax.fori_loop(0, S // 4, rbody, 0)

        def step(i, h, fire_next, guard_recycle):
            buf = bufs[h]
            nbuf = bufs[(h + 1) % NBUF]
            for d in gather_descs(i, buf):
                d.wait()
            if fire_next:
                if guard_recycle:

                    @pl.when(i >= NBUF - 1)
                    def _():
                        out_desc(i + 1 - NBUF, nbuf).wait()

                else:
                    out_desc(i + 1 - NBUF, nbuf).wait()
                for d in gather_descs(i + 1, nbuf):
                    d.start()
            add_pos(buf)
            out_desc(i, buf).start()

        for d in gather_descs(0, b0):
            d.start()

        def body(j, carry):
            for h in range(NBUF):
                step(j * NBUF + h, h, fire_next=True, guard_recycle=True)
            return carry

        lax.fori_loop(0, bpw // NBUF, body, 0)
        for t in range(ntail):
            i = bpw - ntail + t
            step(i, i % NBUF, fire_next=(t + 1 < ntail), guard_recycle=False)
        for t in range(NBUF):
            i = bpw - NBUF + t
            out_desc(i, bufs[i % NBUF]).wait()

    return k(idx.reshape(B * S), table2, pos2)


def kernel(pre_embedding, preembed_mask, embed_table, W, b):
    B, S = pre_embedding.shape
    V, E = embed_table.shape
    w_top = W[:E, :]
    w_bot = W[E:, :]
    pe = jnp.asarray(_sinusoidal_pe(S, E))

    table2 = _table_transform(embed_table, w_top)
    pos2 = _pos_transform(pe, w_bot, b)
    idx = pre_embedding.astype(jnp.int32)
    out = _sc_gather(idx, table2, pos2, B, S, E)
    return (out.reshape(B, S, E), preembed_mask)


# speedup vs baseline: 1.9286x; 1.0072x over previous
import functools

import numpy as np
import jax
import jax.numpy as jnp
from jax import lax
from jax.experimental import pallas as pl
from jax.experimental.pallas import tpu as pltpu
from jax.experimental.pallas import tpu_sc as plsc


def _sinusoidal_pe(seq_len, d_model):
    pos = np.arange(seq_len, dtype=np.float32)[:, None]
    div = np.exp(np.arange(0, d_model, 2, dtype=np.float32) * (-np.log(10000.0) / d_model))
    pe = np.zeros((seq_len, d_model), dtype=np.float32)
    pe[:, 0::2] = np.sin(pos * div)
    pe[:, 1::2] = np.cos(pos * div)
    return pe


def _mm_body(x_ref, w_ref, o_ref):
    o_ref[...] = jnp.dot(x_ref[...], w_ref[...], preferred_element_type=jnp.float32)


def _table_transform(table, w_top):
    v, e = table.shape
    bm = 2000
    assert v % bm == 0
    return pl.pallas_call(
        _mm_body,
        grid=(v // bm,),
        in_specs=[
            pl.BlockSpec((bm, e), lambda i: (i, 0)),
            pl.BlockSpec((e, e), lambda i: (0, 0)),
        ],
        out_specs=pl.BlockSpec((bm, e), lambda i: (i, 0)),
        out_shape=jax.ShapeDtypeStruct((v, e), jnp.float32),
    )(table, w_top)


def _pos_body(pe_ref, w_ref, b_ref, o_ref):
    o_ref[...] = (
        jnp.dot(pe_ref[...], w_ref[...], preferred_element_type=jnp.float32)
        + b_ref[...]
    )


def _pos_transform(pe, w_bot, b):
    s, e = pe.shape
    return pl.pallas_call(
        _pos_body,
        in_specs=[
            pl.BlockSpec((s, e), lambda: (0, 0)),
            pl.BlockSpec((e, e), lambda: (0, 0)),
            pl.BlockSpec((1, e), lambda: (0, 0)),
        ],
        out_specs=pl.BlockSpec((s, e), lambda: (0, 0)),
        out_shape=jax.ShapeDtypeStruct((s, e), jnp.float32),
    )(pe, w_bot, b.reshape(1, e))


def _sc_gather(idx, table2, pos2, B, S, E):
    info = plsc.get_sparse_core_info()
    NC, NS = info.num_cores, info.num_subcores
    NW = NC * NS
    U = S // 2
    UP = 104
    nu = (B * S) // U
    assert nu % NW == 0
    upw = nu // NW
    NBB = 3
    bpw = B // NW
    T = upw

    mesh = plsc.VectorSubcoreMesh(core_axis_name="c", subcore_axis_name="s")

    @functools.partial(
        pl.kernel,
        mesh=mesh,
        out_type=jax.ShapeDtypeStruct((B * S, E), jnp.float32),
        scratch_types=[
            pltpu.VMEM((nu // NW * UP,), jnp.int32),
            pltpu.VMEM((S, E), jnp.float32),
            [pltpu.VMEM((S, E), jnp.float32)] * 3,
            pltpu.SemaphoreType.DMA,
            pltpu.SemaphoreType.DMA,
        ],
    )
    def k(idx_hbm, table2_hbm, pos2_hbm, out_hbm, idx_v, pos_v, bufs, sem_g, sem_w):
        wid = lax.axis_index("s") * NC + lax.axis_index("c")
        base_u = wid * upw
        base_b = wid * bpw
        pltpu.sync_copy(pos2_hbm, pos_v)
        pltpu.sync_copy(idx_hbm.at[pl.ds(base_u * UP, upw * UP)], idx_v)

        def gather_desc(t, buf, half):
            return pltpu.make_async_copy(
                table2_hbm.at[idx_v.at[pl.ds(t * UP, U)]],
                buf.at[pl.ds(half * U, U)],
                sem_g,
            )

        def out_desc(i, buf):
            return pltpu.make_async_copy(
                buf, out_hbm.at[pl.ds((base_b + i) * S, S)], sem_w
            )

        def add_pos(buf, phase):
            def rbody(r4, c2):
                for dr in range(4):
                    r = phase + r4 * 4 + dr
                    for c in range(E // 16):
                        plsc.addupdate(
                            buf.at[r, pl.ds(c * 16, 16)], pos_v[r, pl.ds(c * 16, 16)]
                        )
                return c2

            lax.fori_loop(0, U // 4, rbody, 0)

        def step(t, bi, h, h3, fire, guard_recycle):
            buf = bufs[h3]
            gather_desc(t, buf, h).wait()
            if fire:
                nb3 = (h3 + (h + 3) // 2) % NBB
                nh = (h + 3) % 2
                if nh == 0:
                    rec_i = bi + (h + 3) // 2 - NBB
                    if guard_recycle:

                        @pl.when(rec_i >= 0)
                        def _():
                            out_desc(rec_i, bufs[nb3]).wait()

                    else:
                        out_desc(rec_i, bufs[nb3]).wait()
                gather_desc(t + 3, bufs[nb3], nh).start()
            add_pos(buf, h * U)
            if h == 1:
                out_desc(bi, buf).start()

        gather_desc(0, bufs[0], 0).start()
        gather_desc(1, bufs[0], 1).start()
        gather_desc(2, bufs[1], 0).start()

        def body(j, carry):
            for hh in range(6):
                step(
                    j * 6 + hh,
                    bi=j * 3 + hh // 2,
                    h=hh % 2,
                    h3=(hh // 2) % NBB,
                    fire=True,
                    guard_recycle=True,
                )
            return carry

        nmain = (T - 4) // 6
        lax.fori_loop(0, nmain, body, 0)
        for t in range(nmain * 6, T):
            step(
                t,
                bi=t // 2,
                h=t % 2,
                h3=(t // 2) % NBB,
                fire=(t + 3 < T),
                guard_recycle=False,
            )
        for i in range(bpw - NBB, bpw):
            out_desc(i, bufs[i % NBB]).wait()

    idx_pad = jnp.pad(idx.reshape(nu, U), ((0, 0), (0, UP - U))).reshape(nu * UP)
    return k(idx_pad, table2, pos2)


def kernel(pre_embedding, preembed_mask, embed_table, W, b):
    B, S = pre_embedding.shape
    V, E = embed_table.shape
    w_top = W[:E, :]
    w_bot = W[E:, :]
    pe = jnp.asarray(_sinusoidal_pe(S, E))

    table2 = _table_transform(embed_table, w_top)
    pos2 = _pos_transform(pe, w_bot, b)
    idx = pre_embedding.astype(jnp.int32)
    out = _sc_gather(idx, table2, pos2, B, S, E)
    return (out.reshape(B, S, E), preembed_mask)


# speedup vs baseline: 1.9431x; 1.0075x over previous
import functools

import numpy as np
import jax
import jax.numpy as jnp
from jax import lax
from jax.experimental import pallas as pl
from jax.experimental.pallas import tpu as pltpu
from jax.experimental.pallas import tpu_sc as plsc


def _sinusoidal_pe(seq_len, d_model):
    pos = np.arange(seq_len, dtype=np.float32)[:, None]
    div = np.exp(np.arange(0, d_model, 2, dtype=np.float32) * (-np.log(10000.0) / d_model))
    pe = np.zeros((seq_len, d_model), dtype=np.float32)
    pe[:, 0::2] = np.sin(pos * div)
    pe[:, 1::2] = np.cos(pos * div)
    return pe


def _transform(table, pe, w, b):
    v, e = table.shape
    s = pe.shape[0]
    bm = 2000
    assert v % bm == 0
    nblk = v // bm

    def body(x_ref, pe_ref, w_ref, b_ref, o_ref):
        pid = pl.program_id(0)

        @pl.when(pid < nblk)
        def _():
            o_ref[...] = jnp.dot(
                x_ref[...], w_ref[:e, :], preferred_element_type=jnp.float32
            )

        @pl.when(pid == nblk)
        def _():
            o_ref[:s, :] = (
                jnp.dot(pe_ref[...], w_ref[e:, :], preferred_element_type=jnp.float32)
                + b_ref[...]
            )

    return pl.pallas_call(
        body,
        grid=(nblk + 1,),
        in_specs=[
            pl.BlockSpec((bm, e), lambda i: (jnp.minimum(i, nblk - 1), 0)),
            pl.BlockSpec((s, e), lambda i: (0, 0)),
            pl.BlockSpec((2 * e, e), lambda i: (0, 0)),
            pl.BlockSpec((1, e), lambda i: (0, 0)),
        ],
        out_specs=pl.BlockSpec((bm, e), lambda i: (i, 0)),
        out_shape=jax.ShapeDtypeStruct((v + bm, e), jnp.float32),
    )(table, pe, w, b.reshape(1, e))


def _sc_gather(idx, table2, V, B, S, E):
    info = plsc.get_sparse_core_info()
    NC, NS = info.num_cores, info.num_subcores
    NW = NC * NS
    U = S // 2
    UP = 104
    nu = (B * S) // U
    assert nu % NW == 0
    upw = nu // NW
    NBB = 3
    bpw = B // NW
    T = upw

    mesh = plsc.VectorSubcoreMesh(core_axis_name="c", subcore_axis_name="s")

    @functools.partial(
        pl.kernel,
        mesh=mesh,
        out_type=jax.ShapeDtypeStruct((B * S, E), jnp.float32),
        scratch_types=[
            pltpu.VMEM((nu // NW * UP,), jnp.int32),
            pltpu.VMEM((S, E), jnp.float32),
            [pltpu.VMEM((S, E), jnp.float32)] * 3,
            pltpu.SemaphoreType.DMA,
            pltpu.SemaphoreType.DMA,
        ],
    )
    def k(idx_hbm, table2_hbm, out_hbm, idx_v, pos_v, bufs, sem_g, sem_w):
        wid = lax.axis_index("s") * NC + lax.axis_index("c")
        base_u = wid * upw
        base_b = wid * bpw
        pltpu.sync_copy(table2_hbm.at[pl.ds(V, S)], pos_v)
        pltpu.sync_copy(idx_hbm.at[pl.ds(base_u * UP, upw * UP)], idx_v)

        def gather_desc(t, buf, half):
            return pltpu.make_async_copy(
                table2_hbm.at[idx_v.at[pl.ds(t * UP, U)]],
                buf.at[pl.ds(half * U, U)],
                sem_g,
            )

        def out_desc(i, buf):
            return pltpu.make_async_copy(
                buf, out_hbm.at[pl.ds((base_b + i) * S, S)], sem_w
            )

        def add_pos(buf, phase):
            def rbody(r4, c2):
                for dr in range(4):
                    r = phase + r4 * 4 + dr
                    for c in range(E // 16):
                        plsc.addupdate(
                            buf.at[r, pl.ds(c * 16, 16)], pos_v[r, pl.ds(c * 16, 16)]
                        )
                return c2

            lax.fori_loop(0, U // 4, rbody, 0)

        def step(t, bi, h, h3, fire, guard_recycle):
            buf = bufs[h3]
            gather_desc(t, buf, h).wait()
            if fire:
                nb3 = (h3 + (h + 3) // 2) % NBB
                nh = (h + 3) % 2
                if nh == 0:
                    rec_i = bi + (h + 3) // 2 - NBB
                    if guard_recycle:

                        @pl.when(rec_i >= 0)
                        def _():
                            out_desc(rec_i, bufs[nb3]).wait()

                    else:
                        out_desc(rec_i, bufs[nb3]).wait()
                gather_desc(t + 3, bufs[nb3], nh).start()
            add_pos(buf, h * U)
            if h == 1:
                out_desc(bi, buf).start()

        gather_desc(0, bufs[0], 0).start()
        gather_desc(1, bufs[0], 1).start()
        gather_desc(2, bufs[1], 0).start()

        def body(j, carry):
            for hh in range(6):
                step(
                    j * 6 + hh,
                    bi=j * 3 + hh // 2,
                    h=hh % 2,
                    h3=(hh // 2) % NBB,
                    fire=True,
                    guard_recycle=True,
                )
            return carry

        nmain = (T - 4) // 6
        lax.fori_loop(0, nmain, body, 0)
        for t in range(nmain * 6, T):
            step(
                t,
                bi=t // 2,
                h=t % 2,
                h3=(t // 2) % NBB,
                fire=(t + 3 < T),
                guard_recycle=False,
            )
        for i in range(bpw - NBB, bpw):
            out_desc(i, bufs[i % NBB]).wait()

    idx_pad = jnp.pad(idx.reshape(nu, U), ((0, 0), (0, UP - U))).reshape(nu * UP)
    return k(idx_pad, table2)


def kernel(pre_embedding, preembed_mask, embed_table, W, b):
    B, S = pre_embedding.shape
    V, E = embed_table.shape
    pe = jnp.asarray(_sinusoidal_pe(S, E))

    table2 = _transform(embed_table, pe, W, b)
    idx = pre_embedding.astype(jnp.int32)
    out = _sc_gather(idx, table2, V, B, S, E)
    return (out.reshape(B, S, E), preembed_mask)
